# baseline (speedup 1.0000x reference)
import functools

import jax
import jax.numpy as jnp
from jax import lax
from jax.experimental import pallas as pl
from jax.experimental.pallas import tpu as pltpu
from jax.experimental.pallas import tpu_sc as plsc

VOCAB_SIZE = 1000000
EMBED = 64
BATCH = 4096
SEQ = 50
N = BATCH * SEQ
NC = 2
NS = 16
NW = NC * NS
CHUNK = 128
PER_W = N // NW
NCH = PER_W // CHUNK

_mesh = plsc.VectorSubcoreMesh(core_axis_name="c", subcore_axis_name="s")


@functools.partial(
    pl.kernel,
    mesh=_mesh,
    compiler_params=pltpu.CompilerParams(use_tc_tiling_on_sc=False),
    out_type=jax.ShapeDtypeStruct((N, EMBED), jnp.float32),
    scratch_types=[
        pltpu.VMEM((NCH, CHUNK), jnp.int32),
        pltpu.VMEM((CHUNK, EMBED), jnp.float32),
        pltpu.SemaphoreType.DMA,
    ],
)
def _gather(table_hbm, idx_hbm, out_hbm, idx_v, rows_v, sem):
    wid = lax.axis_index("s") * NC + lax.axis_index("c")
    base = wid * PER_W
    pltpu.sync_copy(idx_hbm.at[wid], idx_v)

    def body(j, carry):
        pltpu.async_copy(table_hbm.at[idx_v.at[j]], rows_v, sem).wait()
        pltpu.sync_copy(rows_v, out_hbm.at[pl.ds(base + j * CHUNK, CHUNK)])
        return carry

    lax.fori_loop(0, NCH, body, 0)


def kernel(inputs, word_embeddings):
    idx = inputs.astype(jnp.int32).reshape(NW, NCH, CHUNK)
    out = _gather(word_embeddings, idx)
    return out.reshape(BATCH, SEQ, EMBED)


# speedup vs baseline: 1.0454x; 1.0454x over previous
import functools

import jax
import jax.numpy as jnp
from jax import lax
from jax.experimental import pallas as pl
from jax.experimental.pallas import tpu as pltpu
from jax.experimental.pallas import tpu_sc as plsc

VOCAB_SIZE = 1000000
EMBED = 64
BATCH = 4096
SEQ = 50
N = BATCH * SEQ
NC = 2
NS = 16
NW = NC * NS
CHUNK = 128
PER_W = N // NW
NCH = PER_W // CHUNK

_mesh = plsc.VectorSubcoreMesh(core_axis_name="c", subcore_axis_name="s")


NBUF = 5


@functools.partial(
    pl.kernel,
    mesh=_mesh,
    compiler_params=pltpu.CompilerParams(use_tc_tiling_on_sc=False),
    out_type=jax.ShapeDtypeStruct((N, EMBED), jnp.float32),
    scratch_types=[
        pltpu.VMEM((NCH, CHUNK), jnp.int32),
        *([pltpu.VMEM((CHUNK, EMBED), jnp.float32)] * NBUF),
        *([pltpu.SemaphoreType.DMA] * NBUF),
    ],
)
def _gather(table_hbm, idx_hbm, out_hbm, idx_v, *bufs_and_sems):
    bufs = bufs_and_sems[:NBUF]
    sems = bufs_and_sems[NBUF:]
    wid = lax.axis_index("s") * NC + lax.axis_index("c")
    base = wid * PER_W
    pltpu.sync_copy(idx_hbm.at[wid], idx_v)

    for b in range(NBUF):
        pltpu.async_copy(table_hbm.at[idx_v.at[b]], bufs[b], sems[b])

    def body(g, carry):
        j0 = g * NBUF
        for b in range(NBUF):
            jj = j0 + b
            pltpu.make_async_copy(
                table_hbm.at[idx_v.at[jj]], bufs[b], sems[b]
            ).wait()
            pltpu.sync_copy(
                bufs[b], out_hbm.at[pl.ds(base + jj * CHUNK, CHUNK)]
            )
            nxt = jj + NBUF

            @pl.when(nxt < NCH)
            def _():
                pltpu.async_copy(table_hbm.at[idx_v.at[nxt]], bufs[b], sems[b])

        return carry

    lax.fori_loop(0, NCH // NBUF, body, 0)


def kernel(inputs, word_embeddings):
    idx = inputs.astype(jnp.int32).reshape(NW, NCH, CHUNK)
    out = _gather(word_embeddings, idx)
    return out.reshape(BATCH, SEQ, EMBED)
